# baseline (speedup 1.0000x reference)
import functools

import jax
import jax.numpy as jnp
from jax import lax
from jax.experimental import pallas as pl
from jax.experimental.pallas import tpu as pltpu
from jax.experimental.pallas import tpu_sc as plsc

NVX = 128
NVY = 128
VIEWS = 180
NDETU = 512
EXTENT = 2
CHANNEL = 8
K = VIEWS * NDETU
NVOX = NVX * NVY
SEG = VIEWS * EXTENT
SCALE = 2.0 * 3.14159265358979323846 / (2.0 * VIEWS * EXTENT)

NPAIR = CHANNEL // 2
NRANGES = 8
VPR = NVOX // NRANGES
GVOX = 16
NGROUPS = VPR // GVOX
CHUNK = GVOX * SEG
NBUF = 4


def _bp_kernel(x_hbm, idx_hbm, out_hbm, table_v, b0, b1, b2, b3, outa_v,
               outb_v, s0, s1, s2, s3):
    c = lax.axis_index("c")
    s = lax.axis_index("s")
    wid = s * 2 + c
    pair = wid % NPAIR
    rng = wid // NPAIR
    tile_vox0 = rng * VPR
    idx_base = tile_vox0 * SEG

    pltpu.sync_copy(x_hbm.at[pair], table_v)

    bufs = (b0, b1, b2, b3)
    sems = (s0, s1, s2, s3)

    def start_fetch(g, b):
        pltpu.make_async_copy(
            idx_hbm.at[pl.ds(idx_base + g * CHUNK, CHUNK)], bufs[b], sems[b]
        ).start()

    def wait_fetch(g, b):
        pltpu.make_async_copy(
            idx_hbm.at[pl.ds(idx_base + g * CHUNK, CHUNK)], bufs[b], sems[b]
        ).wait()

    for g0 in range(NBUF - 1):
        start_fetch(g0, g0)

    lane = jax.lax.iota(jnp.int32, 16)

    def compute_group(g, buf):
        pos0 = lane * SEG + lane

        def gather_step(pos, accs):
            acca, accb = accs
            packed = plsc.load_gather(table_v,
                                      [plsc.load_gather(buf, [pos])])
            va = plsc.bitcast(
                lax.shift_left(packed, jnp.int32(16)), jnp.float32)
            vb = plsc.bitcast(packed & jnp.int32(-65536), jnp.float32)
            return (acca + va, accb + vb)

        def j_body(j, accs):
            j8 = j * 8
            for u in range(8):
                accs = gather_step(pos0 + (j8 + u), accs)
            return accs

        z = jnp.zeros(16, jnp.float32)
        accs = lax.fori_loop(0, 344 // 8, j_body, (z, z))
        for j in range(344, SEG):
            wrap = jnp.where(lane >= SEG - j, SEG, 0)
            accs = gather_step(pos0 + j - wrap, accs)
        acca, accb = accs
        off = g * GVOX
        outa_v[pl.ds(off, 16)] = acca * SCALE
        outb_v[pl.ds(off, 16)] = accb * SCALE

    def ring_body(k, _):
        for b in range(NBUF):
            g = k * NBUF + b
            wait_fetch(g, b)

            @pl.when(g + NBUF - 1 < NGROUPS)
            def _():
                start_fetch(g + NBUF - 1, (b + NBUF - 1) % NBUF)

            compute_group(g, bufs[b])
        return 0

    lax.fori_loop(0, NGROUPS // NBUF, ring_body, 0)

    pltpu.sync_copy(outa_v, out_hbm.at[pair * 2, pl.ds(tile_vox0, VPR)])
    pltpu.sync_copy(outb_v, out_hbm.at[pair * 2 + 1, pl.ds(tile_vox0, VPR)])


@jax.jit
def _backproj(xp, indices):
    f = functools.partial(
        pl.kernel,
        mesh=plsc.VectorSubcoreMesh(core_axis_name="c", subcore_axis_name="s"),
        out_type=jax.ShapeDtypeStruct((CHANNEL, NVOX), jnp.float32),
        compiler_params=pltpu.CompilerParams(needs_layout_passes=False),
        scratch_types=[
            pltpu.VMEM((K,), jnp.int32),
            pltpu.VMEM((CHUNK,), jnp.int32),
            pltpu.VMEM((CHUNK,), jnp.int32),
            pltpu.VMEM((CHUNK,), jnp.int32),
            pltpu.VMEM((CHUNK,), jnp.int32),
            pltpu.VMEM((VPR,), jnp.float32),
            pltpu.VMEM((VPR,), jnp.float32),
            pltpu.SemaphoreType.DMA,
            pltpu.SemaphoreType.DMA,
            pltpu.SemaphoreType.DMA,
            pltpu.SemaphoreType.DMA,
        ],
    )(_bp_kernel)
    return f(xp, indices)


def kernel(input, indices):
    u = input.reshape(CHANNEL, K).view(jnp.uint32)
    half = jnp.uint32(0x8000)
    lo = (u[0::2] + half) >> 16
    hi = (u[1::2] + half) & jnp.uint32(0xFFFF0000)
    packed = (lo | hi).view(jnp.int32)
    out = _backproj(packed, indices)
    return out.reshape(1, CHANNEL, NVX, NVY)
